# baseline (speedup 1.0000x reference)
import functools

import jax
import jax.numpy as jnp
from jax import lax
from jax.experimental import pallas as pl
from jax.experimental.pallas import tpu as pltpu
from jax.experimental.pallas import tpu_sc as plsc

_B = 16384
_D = 32
_NC = 2
_NS = 16
_L = 16
_NW = _NC * _NS
_BPW = _B // _NW
_CHUNK = 128
_NCHUNK = _BPW // _CHUNK


def _sc_body(table_hbm, idx_hbm, out_hbm, idx_v, rows_v, w0_v, sem):
    wid = lax.axis_index("s") * _NC + lax.axis_index("c")
    base = wid * _BPW

    pltpu.sync_copy(idx_hbm.at[pl.ds(base, _BPW)], idx_v)
    pltpu.sync_copy(table_hbm.at[pl.ds(0, 1)], w0_v)

    copies = [
        pltpu.async_copy(
            table_hbm.at[idx_v.at[pl.ds(j * _CHUNK, _CHUNK)]],
            rows_v.at[pl.ds(j * _CHUNK, _CHUNK)],
            sem,
        )
        for j in range(_NCHUNK)
    ]
    for cp in copies:
        cp.wait()

    w0a = w0_v[0, pl.ds(0, _L)]
    w0b = w0_v[0, pl.ds(_L, _L)]

    def splat(vec, lane):
        return lax.gather(
            vec,
            jnp.full((_L, 1), lane, jnp.int32),
            lax.GatherDimensionNumbers(
                offset_dims=(), collapsed_slice_dims=(0,), start_index_map=(0,)
            ),
            slice_sizes=(1,),
            mode=lax.GatherScatterMode.PROMISE_IN_BOUNDS,
        )

    def group_body(g, carry):
        idx16 = idx_v[pl.ds(g * _L, _L)]
        m16 = jnp.where(idx16 != 0, jnp.float32(1.0), jnp.float32(0.0))
        for j in range(_L):
            r = g * _L + j
            msp = splat(m16, j)
            rows_v[r, pl.ds(0, _L)] = rows_v[r, pl.ds(0, _L)] * msp + w0a
            rows_v[r, pl.ds(_L, _L)] = rows_v[r, pl.ds(_L, _L)] * msp + w0b
        return carry

    lax.fori_loop(0, _BPW // _L, group_body, 0)

    pltpu.sync_copy(rows_v, out_hbm.at[pl.ds(base, _BPW)])


@jax.jit
def kernel(user_identifiers, user_embedding_weight):
    mesh = plsc.VectorSubcoreMesh(core_axis_name="c", subcore_axis_name="s")
    run = pl.kernel(
        _sc_body,
        out_type=jax.ShapeDtypeStruct((_B, _D), jnp.float32),
        mesh=mesh,
        scratch_types=[
            pltpu.VMEM((_BPW,), jnp.int32),
            pltpu.VMEM((_BPW, _D), jnp.float32),
            pltpu.VMEM((1, _D), jnp.float32),
            pltpu.SemaphoreType.DMA,
        ],
        compiler_params=pltpu.CompilerParams(use_tc_tiling_on_sc=False),
    )
    out = run(user_embedding_weight, user_identifiers)
    return out.reshape(_B, 1, _D)
